# baseline (speedup 1.0000x reference)
import jax
import jax.numpy as jnp
from jax import lax
from jax.experimental import pallas as pl
from jax.experimental.pallas import tpu as pltpu
from jax.experimental.pallas import tpu_sc as plsc

BATCH = 2048
IN_FEATURES = 1024
WINDOW = 128
NK = 56
NUM_WINDOWS = 14
LANES = 16
CHUNKS = WINDOW // LANES

NUM_CORES = 2
NUM_SUBCORES = 16
NUM_WORKERS = NUM_CORES * NUM_SUBCORES
ROWS_PER_WORKER = BATCH // NUM_WORKERS
ROW_GROUPS = ROWS_PER_WORKER // LANES


def _fbl_body(x_hbm, w_hbm, out_hbm, x_v, w_v, out_v, st0, st1, st2, st3):
    wid = lax.axis_index("s") * NUM_CORES + lax.axis_index("c")
    base = wid * ROWS_PER_WORKER

    pltpu.sync_copy(x_hbm.at[pl.ds(base, ROWS_PER_WORKER)], x_v)
    pltpu.sync_copy(w_hbm, w_v)

    lanes = lax.iota(jnp.int32, LANES)
    stage = [st0, st1, st2, st3]

    def step(t, _):
        j = t // ROW_GROUPS
        bi = t % ROW_GROUPS
        start = lax.select(
            j == NUM_WINDOWS - 1,
            jnp.int32(IN_FEATURES - WINDOW),
            jnp.int32(64) * j,
        )
        wv = [
            [w_v[4 * j + c, pl.ds(16 * k, LANES)] for k in range(CHUNKS)]
            for c in range(4)
        ]
        b0 = bi * LANES
        for bl in range(LANES):
            xv = [
                x_v[b0 + bl, pl.ds(start + 16 * k, LANES)]
                for k in range(CHUNKS)
            ]
            for c in range(4):
                acc = xv[0] * wv[c][0]
                for k in range(1, CHUNKS):
                    acc = acc + xv[k] * wv[c][k]
                stage[c][pl.ds(bl * LANES, LANES)] = acc
        for c in range(4):
            tot = plsc.load_gather(stage[c], [lanes * LANES])
            for l in range(1, LANES):
                tot = tot + plsc.load_gather(stage[c], [lanes * LANES + l])
            idx = lanes * NK + (b0 * NK + 4 * j + c)
            plsc.store_scatter(out_v, [idx], tot)
        return _

    lax.fori_loop(0, NUM_WINDOWS * ROW_GROUPS, step, None)

    pltpu.sync_copy(
        out_v, out_hbm.at[pl.ds(base * NK, ROWS_PER_WORKER * NK)]
    )


@jax.jit
def _fbl(x, weight):
    mesh = plsc.VectorSubcoreMesh(
        core_axis_name="c",
        subcore_axis_name="s",
        num_cores=NUM_CORES,
        num_subcores=NUM_SUBCORES,
    )
    run = pl.kernel(
        _fbl_body,
        out_type=jax.ShapeDtypeStruct((BATCH * NK,), jnp.float32),
        mesh=mesh,
        scratch_types=[
            pltpu.VMEM((ROWS_PER_WORKER, IN_FEATURES), jnp.float32),
            pltpu.VMEM((NK, WINDOW), jnp.float32),
            pltpu.VMEM((ROWS_PER_WORKER * NK,), jnp.float32),
            pltpu.VMEM((LANES * LANES,), jnp.float32),
            pltpu.VMEM((LANES * LANES,), jnp.float32),
            pltpu.VMEM((LANES * LANES,), jnp.float32),
            pltpu.VMEM((LANES * LANES,), jnp.float32),
        ],
        compiler_params=pltpu.CompilerParams(needs_layout_passes=False),
    )
    return run(x, weight).reshape(BATCH, NK)


def kernel(x, weight, fbank):
    del fbank
    return _fbl(x, weight)
